# Initial kernel scaffold; baseline (speedup 1.0000x reference)
import jax
import jax.numpy as jnp
from jax.experimental import pallas as pl


def kernel(x_sparse, domain_indicator, tables, W1, b1, TW1, Tb1, TW2, Tb2):
    raise NotImplementedError("write your pallas kernel here")


# speedup vs baseline: 1.9144x; 1.9144x over previous
import functools

import jax
import jax.numpy as jnp
from jax import lax
from jax.experimental import pallas as pl
from jax.experimental.pallas import tpu as pltpu
from jax.experimental.pallas import tpu_sc as plsc

B = 16384
N_FIELDS = 26
VOCAB = 100000
EMB = 16
DOMAINS = 4
BOTTOM = 128
TOWER = 8

NC = 2
NS = 16
NW = NC * NS

ROWS = B * N_FIELDS
RPW = ROWS // NW
CH = 128
G = 13
GROUP_ROWS = G * CH
NG = RPW // GROUP_ROWS


def _sc_gather_body(idx_hbm, table_hbm, out_hbm, idx_v, rows_v, gsem):
    wid = lax.axis_index("s") * NC + lax.axis_index("c")
    pltpu.sync_copy(idx_hbm.at[wid], idx_v)

    def group(g, carry):
        for j in range(G):
            pltpu.async_copy(
                table_hbm.at[idx_v.at[g, j]],
                rows_v.at[pl.ds(j * CH, CH)],
                gsem,
            )
        pltpu.make_async_copy(out_hbm.at[wid, g], rows_v, gsem).wait()
        pltpu.sync_copy(rows_v, out_hbm.at[wid, g])
        return carry

    lax.fori_loop(0, NG, group, 0)


@functools.cache
def _make_sc_gather():
    return pl.kernel(
        _sc_gather_body,
        out_type=jax.ShapeDtypeStruct((NW, NG, GROUP_ROWS, EMB), jnp.float32),
        mesh=plsc.VectorSubcoreMesh(
            core_axis_name="c", subcore_axis_name="s",
            num_cores=NC, num_subcores=NS),
        scratch_types=[
            pltpu.VMEM((NG, G, CH), jnp.int32),
            pltpu.VMEM((GROUP_ROWS, EMB), jnp.float32),
            pltpu.SemaphoreType.DMA,
        ],
        compiler_params=pltpu.CompilerParams(use_tc_tiling_on_sc=False),
    )


BLK = 1024
NBLK = B // BLK


def _tc_body(flat_ref, dom_ref, w1_ref, b1_ref, tw1_ref, tb1_ref, tw2_ref,
             tb2_ref, out_ref):
    flat = flat_ref[...]
    h = jnp.dot(flat, w1_ref[...], preferred_element_type=jnp.float32)
    h = jnp.maximum(h + b1_ref[...], 0.0)
    t = jnp.dot(h, tw1_ref[...], preferred_element_type=jnp.float32)
    t = jnp.maximum(t + tb1_ref[...], 0.0)
    logits = jnp.dot(t, tw2_ref[...], preferred_element_type=jnp.float32)
    logits = logits + tb2_ref[...]
    dom = dom_ref[...].reshape(BLK, 1)
    onehot = dom == lax.broadcasted_iota(jnp.int32, (1, DOMAINS), 1)
    logit = jnp.sum(jnp.where(onehot, logits, 0.0), axis=1)
    out_ref[...] = (1.0 / (1.0 + jnp.exp(-logit))).reshape(1, 1, BLK)


def kernel(x_sparse, domain_indicator, tables, W1, b1, TW1, Tb1, TW2, Tb2):
    offs = (jnp.arange(N_FIELDS, dtype=jnp.int32) * VOCAB)[None, :]
    flat_idx = (x_sparse + offs).reshape(NW, NG, G, CH)
    table2d = tables.reshape(N_FIELDS * VOCAB, EMB)

    gathered = _make_sc_gather()(flat_idx, table2d)
    flat = gathered.reshape(B, N_FIELDS * EMB)

    tw1c = TW1.transpose(1, 0, 2).reshape(BOTTOM, DOMAINS * TOWER)
    tb1c = Tb1.reshape(1, DOMAINS * TOWER)
    eye = jnp.eye(DOMAINS, dtype=TW2.dtype)
    tw2bd = (TW2[:, :, 0][:, :, None] * eye[:, None, :]).reshape(
        DOMAINS * TOWER, DOMAINS)
    tb2c = Tb2.reshape(1, DOMAINS)

    dom3 = domain_indicator.reshape(NBLK, 1, BLK)

    out3 = pl.pallas_call(
        _tc_body,
        grid=(NBLK,),
        in_specs=[
            pl.BlockSpec((BLK, N_FIELDS * EMB), lambda i: (i, 0)),
            pl.BlockSpec((1, 1, BLK), lambda i: (i, 0, 0)),
            pl.BlockSpec((N_FIELDS * EMB, BOTTOM), lambda i: (0, 0)),
            pl.BlockSpec((1, BOTTOM), lambda i: (0, 0)),
            pl.BlockSpec((BOTTOM, DOMAINS * TOWER), lambda i: (0, 0)),
            pl.BlockSpec((1, DOMAINS * TOWER), lambda i: (0, 0)),
            pl.BlockSpec((DOMAINS * TOWER, DOMAINS), lambda i: (0, 0)),
            pl.BlockSpec((1, DOMAINS), lambda i: (0, 0)),
        ],
        out_specs=pl.BlockSpec((1, 1, BLK), lambda i: (i, 0, 0)),
        out_shape=jax.ShapeDtypeStruct((NBLK, 1, BLK), jnp.float32),
        compiler_params=pltpu.CompilerParams(
            dimension_semantics=("arbitrary",),
        ),
    )(flat, dom3, W1, b1.reshape(1, BOTTOM), tw1c, tb1c, tw2bd, tb2c)

    return out3.reshape(B)
